# baseline (speedup 1.0000x reference)
import functools

import jax
import jax.numpy as jnp
from jax import lax
from jax.experimental import pallas as pl
from jax.experimental.pallas import tpu as pltpu
from jax.experimental.pallas import tpu_sc as plsc

N = 10000
E = 160000
D = 256
H = 128
WIDTH = 144
NC = 2
NS = 16
EPT = E // NS
CH = 128
HCH = CH // 2
RPT = 640
RCH = 80
BLK = 2000


def _leaky(x):
    return jnp.where(x >= 0.0, x, 0.2 * x)


def _tc_body(node_ref, w_ref, b_ref, a_ref, wext_ref, sin_ref, sout_ref,
             mi_ref, mo_ref):
    i = pl.program_id(0)
    x = node_ref[...]
    wn = jnp.dot(x, w_ref[...], preferred_element_type=jnp.float32) + b_ref[...]
    a1 = a_ref[0:D, :]
    a2 = a_ref[D:2 * D, :]
    sin = jnp.dot(wn, a1, preferred_element_type=jnp.float32)
    sout = jnp.dot(wn, a2, preferred_element_type=jnp.float32)
    lane = lax.broadcasted_iota(jnp.int32, (BLK, WIDTH - H), 1)
    ones_col = jnp.where(lane == 0, 1.0, 0.0).astype(jnp.float32)
    h0 = jnp.concatenate([wn[:, :H], ones_col], axis=1)
    h1 = jnp.concatenate([wn[:, H:], ones_col], axis=1)
    wext_ref[...] = jnp.stack([h0, h1], axis=0)
    sin_ref[...] = sin
    sout_ref[...] = sout
    mi = jnp.max(sin)
    mo = jnp.max(sout)
    mi_ref[...] = jnp.where(i == 0, mi, jnp.maximum(mi_ref[...], mi))
    mo_ref[...] = jnp.where(i == 0, mo, jnp.maximum(mo_ref[...], mo))


def _tc_phase(node, W, b2, a_kernel):
    grid = (N // BLK,)
    return pl.pallas_call(
        _tc_body,
        grid=grid,
        in_specs=[
            pl.BlockSpec((BLK, D), lambda i: (i, 0)),
            pl.BlockSpec((D, D), lambda i: (0, 0)),
            pl.BlockSpec((1, D), lambda i: (0, 0)),
            pl.BlockSpec((2 * D, 1), lambda i: (0, 0)),
        ],
        out_specs=[
            pl.BlockSpec((NC, BLK, WIDTH), lambda i: (0, i, 0)),
            pl.BlockSpec((BLK, 1), lambda i: (i, 0)),
            pl.BlockSpec((BLK, 1), lambda i: (i, 0)),
            pl.BlockSpec((1, 1), lambda i: (0, 0)),
            pl.BlockSpec((1, 1), lambda i: (0, 0)),
        ],
        out_shape=[
            jax.ShapeDtypeStruct((NC, N, WIDTH), jnp.float32),
            jax.ShapeDtypeStruct((N, 1), jnp.float32),
            jax.ShapeDtypeStruct((N, 1), jnp.float32),
            jax.ShapeDtypeStruct((1, 1), jnp.float32),
            jax.ShapeDtypeStruct((1, 1), jnp.float32),
        ],
    )(node, W, b2, a_kernel)


def _drain(src, dst, sem):
    pltpu.make_async_copy(src, dst, sem).wait()


def _sc_body(wext_hbm, sin_hbm, sout_hbm, ii_hbm, io_hbm, c_hbm, out_hbm,
             accum, sin_tbl, sout_tbl, iiP0, iiP1, ioP0, ioP1,
             s0, s1, s2, s3, g0, g1, g2, g3,
             ev_v, rows_v, ctbl, iit, iot,
             ps0, ps1, gq0, gq1, gq2, gq3, sq0, sq1, sq2, sq3):
    sidxs = (s0, s1, s2, s3)
    gidxs = (g0, g1, g2, g3)
    gss = (gq0, gq1, gq2, gq3)
    sss = (sq0, sq1, sq2, sq3)
    c = lax.axis_index("c")
    s = lax.axis_index("s")
    pltpu.sync_copy(sin_hbm, sin_tbl)
    pltpu.sync_copy(sout_hbm, sout_tbl)
    pltpu.sync_copy(c_hbm, ctbl)

    def zrow(r, carry):
        for j in range(WIDTH // 16):
            rows_v[r, pl.ds(j * 16, 16)] = jnp.zeros((16,), jnp.float32)
        return carry
    lax.fori_loop(0, RCH, zrow, 0)
    nch = jnp.where(s == NS - 1, (N - (NS - 1) * RPT) // RCH, RPT // RCH)

    def zchunk(t, carry):
        pltpu.sync_copy(rows_v.at[pl.ds(0, RCH)],
                        accum.at[pl.ds(s * RPT + t * RCH, RCH)])
        return carry
    lax.fori_loop(0, nch, zchunk, 0)
    plsc.subcore_barrier()

    cvec = ctbl[...]
    ebase = s * EPT
    coff = c * N
    idxbufs = ((iiP0, ioP0, ps0), (iiP1, ioP1, ps1))
    NFULL = EPT // CH
    QH = CH // 4
    quarters = tuple((t * QH, gss[t], sss[t], sidxs[t], gidxs[t])
                     for t in range(4))
    onehot = jnp.where(lax.iota(jnp.int32, 16) == 0, 1.0, 0.0)

    def prefetch(q, p):
        off = pl.multiple_of(ebase + q * CH, 8)
        pltpu.async_copy(ii_hbm.at[pl.ds(off, CH)], idxbufs[p][0], idxbufs[p][2])
        pltpu.async_copy(io_hbm.at[pl.ds(off, CH)], idxbufs[p][1], idxbufs[p][2])

    def wait_idx(p):
        _drain(ii_hbm.at[pl.ds(0, CH)], idxbufs[p][0], idxbufs[p][2])
        _drain(io_hbm.at[pl.ds(0, CH)], idxbufs[p][1], idxbufs[p][2])

    def build_and_gather(p):
        iiP, ioP, _ = idxbufs[p]
        for (q0, gs, ss, sidx, gidx) in quarters:
            for g in range(QH // 16):
                sl = pl.ds(g * 16, 16)
                slsrc = pl.ds(q0 + g * 16, 16)
                gidx[sl] = ioP[slsrc] + coff
                sidx[sl] = iiP[slsrc]
            pltpu.async_copy(wext_hbm.at[gidx], rows_v.at[pl.ds(q0, QH)], gs)

    def compute_ev(p):
        iiP, ioP, _ = idxbufs[p]
        for g in range(CH // 16):
            sl = pl.ds(g * 16, 16)
            a = (plsc.load_gather(sin_tbl, [iiP[sl]])
                 + plsc.load_gather(sout_tbl, [ioP[sl]]))
            ev_v[sl] = jnp.exp(_leaky(a) - cvec)

    def scale_quarter(q0):
        def _grp(g, carry3):
            g0 = g * 16
            ev16 = ev_v[pl.ds(q0 + g0, 16)]
            for l in range(16):
                eb = jnp.full((16,), ev16[l], jnp.float32)
                r = q0 + g0 + l
                for j in range(H // 16):
                    slj = pl.ds(j * 16, 16)
                    rows_v[r, slj] = rows_v[r, slj] * eb
                rows_v[r, pl.ds(H, 16)] = eb * onehot
            return carry3
        lax.fori_loop(0, QH // 16, _grp, 0)

    prefetch(0, 0)
    wait_idx(0)
    build_and_gather(0)
    prefetch(1, 1)

    def pair_body(m, carry):
        for p in (0, 1):
            q = 2 * m + p
            compute_ev(p)
            for (q0, gs, ss, sidx, gidx) in quarters:
                _drain(wext_hbm.at[gidx], rows_v.at[pl.ds(q0, QH)], gs)
                scale_quarter(q0)
                pltpu.async_copy(rows_v.at[pl.ds(q0, QH)],
                                 accum.at[sidx], ss, add=True)
            @pl.when(m < NFULL // 2 - 1)
            def _():
                prefetch(q + 2, p)
            if p == 0:
                wait_idx(1)
                for (q0, gs, ss, sidx, gidx) in quarters:
                    _drain(rows_v.at[pl.ds(q0, QH)], accum.at[sidx], ss)
                build_and_gather(1)
            else:
                @pl.when(m < NFULL // 2 - 1)
                def _():
                    wait_idx(0)
                    for (q0, gs, ss, sidx, gidx) in quarters:
                        _drain(rows_v.at[pl.ds(q0, QH)], accum.at[sidx], ss)
                    build_and_gather(0)
        return carry
    lax.fori_loop(0, NFULL // 2, pair_body, 0)

    TL = EPT % CH
    for (q0, gs, ss, sidx, gidx) in quarters:
        _drain(rows_v.at[pl.ds(q0, QH)], accum.at[sidx], ss)
    toff = pl.multiple_of(ebase + NFULL * CH, 8)
    pltpu.sync_copy(ii_hbm.at[pl.ds(toff, TL)], iit)
    pltpu.sync_copy(io_hbm.at[pl.ds(toff, TL)], iot)
    a = (plsc.load_gather(sin_tbl, [iit[...]])
         + plsc.load_gather(sout_tbl, [iot[...]]))
    iot[...] = iot[...] + coff
    gt = pltpu.async_copy(wext_hbm.at[iot], rows_v.at[pl.ds(0, TL)], gss[0])
    ev_v[pl.ds(0, TL)] = jnp.exp(_leaky(a) - cvec)
    gt.wait()
    ev16 = ev_v[pl.ds(0, TL)]
    for l in range(TL):
        eb = jnp.full((16,), ev16[l], jnp.float32)
        for j in range(H // 16):
            slj = pl.ds(j * 16, 16)
            rows_v[l, slj] = rows_v[l, slj] * eb
        rows_v[l, pl.ds(H, 16)] = eb * onehot
    pltpu.async_copy(rows_v.at[pl.ds(0, TL)], accum.at[iit], sss[0],
                     add=True).wait()
    plsc.subcore_barrier()

    rbase = s * RPT

    def out_chunk(t, carry):
        r0 = rbase + t * RCH
        pltpu.sync_copy(accum.at[pl.ds(r0, RCH)], rows_v.at[pl.ds(0, RCH)])

        def rb(r, rc):
            den16 = rows_v[r, pl.ds(H, 16)]
            inv16 = 1.0 / (den16 + 1e-16)
            db = jnp.full((16,), inv16[0], jnp.float32)
            for j in range(H // 16):
                slj = pl.ds(j * 16, 16)
                rows_v[r, slj] = _leaky(rows_v[r, slj] * db)
            return rc
        lax.fori_loop(0, RCH, rb, 0)
        pltpu.sync_copy(rows_v.at[pl.ds(0, RCH), pl.ds(0, H)],
                        out_hbm.at[c, pl.ds(r0, RCH)])
        return carry
    lax.fori_loop(0, nch, out_chunk, 0)


@functools.partial(jax.jit, static_argnames=())
def _sc_phase(wext2, sin, sout, idx_in, idx_out, c16):
    mesh = plsc.VectorSubcoreMesh(core_axis_name="c", subcore_axis_name="s",
                                  num_cores=NC, num_subcores=NS)
    f = pl.kernel(
        _sc_body,
        out_type=jax.ShapeDtypeStruct((NC, N, H), jnp.float32),
        mesh=mesh,
        scratch_types=[
            pltpu.VMEM_SHARED((N, WIDTH), jnp.float32),
            pltpu.VMEM((N,), jnp.float32),
            pltpu.VMEM((N,), jnp.float32),
            pltpu.VMEM((CH,), jnp.int32),
            pltpu.VMEM((CH,), jnp.int32),
            pltpu.VMEM((CH,), jnp.int32),
            pltpu.VMEM((CH,), jnp.int32),
            pltpu.VMEM((CH // 4,), jnp.int32),
            pltpu.VMEM((CH // 4,), jnp.int32),
            pltpu.VMEM((CH // 4,), jnp.int32),
            pltpu.VMEM((CH // 4,), jnp.int32),
            pltpu.VMEM((CH // 4,), jnp.int32),
            pltpu.VMEM((CH // 4,), jnp.int32),
            pltpu.VMEM((CH // 4,), jnp.int32),
            pltpu.VMEM((CH // 4,), jnp.int32),
            pltpu.VMEM((CH,), jnp.float32),
            pltpu.VMEM((CH, WIDTH), jnp.float32),
            pltpu.VMEM((16,), jnp.float32),
            pltpu.VMEM((EPT % CH,), jnp.int32),
            pltpu.VMEM((EPT % CH,), jnp.int32),
            pltpu.SemaphoreType.DMA,
            pltpu.SemaphoreType.DMA,
            pltpu.SemaphoreType.DMA,
            pltpu.SemaphoreType.DMA,
            pltpu.SemaphoreType.DMA,
            pltpu.SemaphoreType.DMA,
            pltpu.SemaphoreType.DMA,
            pltpu.SemaphoreType.DMA,
            pltpu.SemaphoreType.DMA,
            pltpu.SemaphoreType.DMA,
        ],
        compiler_params=pltpu.CompilerParams(needs_layout_passes=False,
                                             use_tc_tiling_on_sc=False),
    )
    return f(wext2, sin, sout, idx_in, idx_out, c16)


def kernel(node, edge, edge_index, W, b, a_kernel):
    del edge
    wext, sin, sout, mi, mo = _tc_phase(node, W, b.reshape(1, D), a_kernel)
    cval = _leaky(mi[0, 0] + mo[0, 0])
    c16 = jnp.full((16,), cval, jnp.float32)
    out2 = _sc_phase(
        wext.reshape(NC * N, WIDTH),
        sin.reshape(N),
        sout.reshape(N),
        edge_index[:, 0],
        edge_index[:, 1],
        c16,
    )
    return jnp.concatenate([out2[0], out2[1]], axis=1)


# speedup vs baseline: 1.1539x; 1.1539x over previous
import functools

import jax
import jax.numpy as jnp
from jax import lax
from jax.experimental import pallas as pl
from jax.experimental.pallas import tpu as pltpu
from jax.experimental.pallas import tpu_sc as plsc

N = 10000
E = 160000
D = 256
H = 128
WIDTH = 144
NC = 2
NS = 16
EPT = E // NS
CH = 128
HCH = CH // 2
RPT = 640
RCH = 80
BLK = 2000


def _leaky(x):
    return jnp.where(x >= 0.0, x, 0.2 * x)


def _tc_body(node_ref, w_ref, b_ref, a_ref, wext_ref, sin_ref, sout_ref,
             mi_ref, mo_ref):
    i = pl.program_id(0)
    x = node_ref[...]
    wn = jnp.dot(x, w_ref[...], preferred_element_type=jnp.float32) + b_ref[...]
    a1 = a_ref[0:D, :]
    a2 = a_ref[D:2 * D, :]
    sin = jnp.dot(wn, a1, preferred_element_type=jnp.float32)
    sout = jnp.dot(wn, a2, preferred_element_type=jnp.float32)
    lane = lax.broadcasted_iota(jnp.int32, (BLK, WIDTH - H), 1)
    ones_col = jnp.where(lane == 0, 1.0, 0.0).astype(jnp.float32)
    h0 = jnp.concatenate([wn[:, :H], ones_col], axis=1)
    h1 = jnp.concatenate([wn[:, H:], ones_col], axis=1)
    wext_ref[...] = jnp.stack([h0, h1], axis=0)
    sin_ref[...] = sin
    sout_ref[...] = sout
    mi = jnp.max(sin)
    mo = jnp.max(sout)
    mi_ref[...] = jnp.where(i == 0, mi, jnp.maximum(mi_ref[...], mi))
    mo_ref[...] = jnp.where(i == 0, mo, jnp.maximum(mo_ref[...], mo))


def _tc_phase(node, W, b2, a_kernel):
    grid = (N // BLK,)
    return pl.pallas_call(
        _tc_body,
        grid=grid,
        in_specs=[
            pl.BlockSpec((BLK, D), lambda i: (i, 0)),
            pl.BlockSpec((D, D), lambda i: (0, 0)),
            pl.BlockSpec((1, D), lambda i: (0, 0)),
            pl.BlockSpec((2 * D, 1), lambda i: (0, 0)),
        ],
        out_specs=[
            pl.BlockSpec((NC, BLK, WIDTH), lambda i: (0, i, 0)),
            pl.BlockSpec((BLK, 1), lambda i: (i, 0)),
            pl.BlockSpec((BLK, 1), lambda i: (i, 0)),
            pl.BlockSpec((1, 1), lambda i: (0, 0)),
            pl.BlockSpec((1, 1), lambda i: (0, 0)),
        ],
        out_shape=[
            jax.ShapeDtypeStruct((NC, N, WIDTH), jnp.float32),
            jax.ShapeDtypeStruct((N, 1), jnp.float32),
            jax.ShapeDtypeStruct((N, 1), jnp.float32),
            jax.ShapeDtypeStruct((1, 1), jnp.float32),
            jax.ShapeDtypeStruct((1, 1), jnp.float32),
        ],
    )(node, W, b2, a_kernel)


def _drain(src, dst, sem):
    pltpu.make_async_copy(src, dst, sem).wait()


def _sc_body(wext_hbm, sin_hbm, sout_hbm, ii_hbm, io_hbm, c_hbm, out_hbm,
             accum, sin_tbl, sout_tbl, iiP0, iiP1, ioP0, ioP1,
             io2A, io2B, iiA, iiB, ev_v, rows_v, ctbl, iit, iot,
             ps0, ps1, gsA, gsB, ssA, ssB):
    c = lax.axis_index("c")
    s = lax.axis_index("s")
    pltpu.sync_copy(sin_hbm, sin_tbl)
    pltpu.sync_copy(sout_hbm, sout_tbl)
    pltpu.sync_copy(c_hbm, ctbl)

    def zrow(r, carry):
        for j in range(WIDTH // 16):
            rows_v[r, pl.ds(j * 16, 16)] = jnp.zeros((16,), jnp.float32)
        return carry
    lax.fori_loop(0, RCH, zrow, 0)
    nch = jnp.where(s == NS - 1, (N - (NS - 1) * RPT) // RCH, RPT // RCH)

    def zchunk(t, carry):
        pltpu.sync_copy(rows_v.at[pl.ds(0, RCH)],
                        accum.at[pl.ds(s * RPT + t * RCH, RCH)])
        return carry
    lax.fori_loop(0, nch, zchunk, 0)
    plsc.subcore_barrier()

    cvec = ctbl[...]
    ebase = s * EPT
    coff = c * N
    idxbufs = ((iiP0, ioP0, ps0), (iiP1, ioP1, ps1))
    halves = ((0, gsA, ssA, iiA, io2A), (HCH, gsB, ssB, iiB, io2B))
    NFULL = EPT // CH

    def prefetch(q, p):
        off = pl.multiple_of(ebase + q * CH, 8)
        pltpu.async_copy(ii_hbm.at[pl.ds(off, CH)], idxbufs[p][0], idxbufs[p][2])
        pltpu.async_copy(io_hbm.at[pl.ds(off, CH)], idxbufs[p][1], idxbufs[p][2])

    def wait_idx(p):
        _drain(ii_hbm.at[pl.ds(0, CH)], idxbufs[p][0], idxbufs[p][2])
        _drain(io_hbm.at[pl.ds(0, CH)], idxbufs[p][1], idxbufs[p][2])

    onehot = jnp.where(lax.iota(jnp.int32, 16) == 0, 1.0, 0.0)

    def scale_half(h0):
        def _grp(g, carry3):
            g0 = g * 16
            ev16 = ev_v[pl.ds(h0 + g0, 16)]
            for l in range(16):
                eb = jnp.full((16,), ev16[l], jnp.float32)
                r = h0 + g0 + l
                for j in range(H // 16):
                    slj = pl.ds(j * 16, 16)
                    rows_v[r, slj] = rows_v[r, slj] * eb
                rows_v[r, pl.ds(H, 16)] = eb * onehot
            return carry3
        lax.fori_loop(0, HCH // 16, _grp, 0)

    prefetch(0, 0)

    def pair_body(m, carry):
        for p in (0, 1):
            q = 2 * m + p
            iiP, ioP, _ = idxbufs[p]
            wait_idx(p)
            if p == 0:
                prefetch(q + 1, p ^ 1)
            else:
                @pl.when(m < NFULL // 2 - 1)
                def _():
                    prefetch(q + 1, p ^ 1)
            gdescs = []
            for hi, (h0, gs, ss, sidx, gidx) in enumerate(halves):
                if p == 0:
                    @pl.when(m > 0)
                    def _():
                        _drain(rows_v.at[pl.ds(h0, HCH)], accum.at[sidx], ss)
                else:
                    _drain(rows_v.at[pl.ds(h0, HCH)], accum.at[sidx], ss)
                for g in range(HCH // 16):
                    sl = pl.ds(g * 16, 16)
                    slsrc = pl.ds(h0 + g * 16, 16)
                    gidx[sl] = ioP[slsrc] + coff
                    sidx[sl] = iiP[slsrc]
                gdescs.append(pltpu.async_copy(
                    wext_hbm.at[gidx], rows_v.at[pl.ds(h0, HCH)], gs))
            for g in range(CH // 16):
                sl = pl.ds(g * 16, 16)
                a = (plsc.load_gather(sin_tbl, [iiP[sl]])
                     + plsc.load_gather(sout_tbl, [ioP[sl]]))
                ev_v[sl] = jnp.exp(_leaky(a) - cvec)
            for hi, (h0, gs, ss, sidx, gidx) in enumerate(halves):
                gdescs[hi].wait()
                scale_half(h0)
                pltpu.async_copy(rows_v.at[pl.ds(h0, HCH)],
                                 accum.at[sidx], ss, add=True)
        return carry
    lax.fori_loop(0, NFULL // 2, pair_body, 0)

    TL = EPT % CH
    for hi, (h0, gs, ss, sidx, gidx) in enumerate(halves):
        _drain(rows_v.at[pl.ds(h0, HCH)], accum.at[sidx], ss)
    toff = pl.multiple_of(ebase + NFULL * CH, 8)
    pltpu.sync_copy(ii_hbm.at[pl.ds(toff, TL)], iit)
    pltpu.sync_copy(io_hbm.at[pl.ds(toff, TL)], iot)
    a = (plsc.load_gather(sin_tbl, [iit[...]])
         + plsc.load_gather(sout_tbl, [iot[...]]))
    iot[...] = iot[...] + coff
    gt = pltpu.async_copy(wext_hbm.at[iot], rows_v.at[pl.ds(0, TL)], gsA)
    ev_v[pl.ds(0, TL)] = jnp.exp(_leaky(a) - cvec)
    gt.wait()
    ev16 = ev_v[pl.ds(0, TL)]
    for l in range(TL):
        eb = jnp.full((16,), ev16[l], jnp.float32)
        for j in range(WIDTH // 16):
            slj = pl.ds(j * 16, 16)
            rows_v[l, slj] = rows_v[l, slj] * eb
    pltpu.async_copy(rows_v.at[pl.ds(0, TL)], accum.at[iit], ssA,
                     add=True).wait()
    plsc.subcore_barrier()

    rbase = s * RPT

    def out_chunk(t, carry):
        r0 = rbase + t * RCH
        pltpu.sync_copy(accum.at[pl.ds(r0, RCH)], rows_v.at[pl.ds(0, RCH)])

        def rb(r, rc):
            den16 = rows_v[r, pl.ds(H, 16)]
            inv16 = 1.0 / (den16 + 1e-16)
            db = jnp.full((16,), inv16[0], jnp.float32)
            for j in range(H // 16):
                slj = pl.ds(j * 16, 16)
                rows_v[r, slj] = _leaky(rows_v[r, slj] * db)
            return rc
        lax.fori_loop(0, RCH, rb, 0)
        pltpu.sync_copy(rows_v.at[pl.ds(0, RCH), pl.ds(0, H)],
                        out_hbm.at[c, pl.ds(r0, RCH)])
        return carry
    lax.fori_loop(0, nch, out_chunk, 0)


@functools.partial(jax.jit, static_argnames=())
def _sc_phase(wext2, sin, sout, idx_in, idx_out, c16):
    mesh = plsc.VectorSubcoreMesh(core_axis_name="c", subcore_axis_name="s",
                                  num_cores=NC, num_subcores=NS)
    f = pl.kernel(
        _sc_body,
        out_type=jax.ShapeDtypeStruct((NC, N, H), jnp.float32),
        mesh=mesh,
        scratch_types=[
            pltpu.VMEM_SHARED((N, WIDTH), jnp.float32),
            pltpu.VMEM((N,), jnp.float32),
            pltpu.VMEM((N,), jnp.float32),
            pltpu.VMEM((CH,), jnp.int32),
            pltpu.VMEM((CH,), jnp.int32),
            pltpu.VMEM((CH,), jnp.int32),
            pltpu.VMEM((CH,), jnp.int32),
            pltpu.VMEM((HCH,), jnp.int32),
            pltpu.VMEM((HCH,), jnp.int32),
            pltpu.VMEM((HCH,), jnp.int32),
            pltpu.VMEM((HCH,), jnp.int32),
            pltpu.VMEM((CH,), jnp.float32),
            pltpu.VMEM((CH, WIDTH), jnp.float32),
            pltpu.VMEM((16,), jnp.float32),
            pltpu.VMEM((EPT % CH,), jnp.int32),
            pltpu.VMEM((EPT % CH,), jnp.int32),
            pltpu.SemaphoreType.DMA,
            pltpu.SemaphoreType.DMA,
            pltpu.SemaphoreType.DMA,
            pltpu.SemaphoreType.DMA,
            pltpu.SemaphoreType.DMA,
            pltpu.SemaphoreType.DMA,
        ],
        compiler_params=pltpu.CompilerParams(needs_layout_passes=False,
                                             use_tc_tiling_on_sc=False),
    )
    return f(wext2, sin, sout, idx_in, idx_out, c16)


def kernel(node, edge, edge_index, W, b, a_kernel):
    del edge
    wext, sin, sout, mi, mo = _tc_phase(node, W, b.reshape(1, D), a_kernel)
    cval = _leaky(mi[0, 0] + mo[0, 0])
    c16 = jnp.full((16,), cval, jnp.float32)
    out2 = _sc_phase(
        wext.reshape(NC * N, WIDTH),
        sin.reshape(N),
        sout.reshape(N),
        edge_index[:, 0],
        edge_index[:, 1],
        c16,
    )
    return jnp.concatenate([out2[0], out2[1]], axis=1)
